# Initial kernel scaffold; baseline (speedup 1.0000x reference)
import jax
import jax.numpy as jnp
from jax.experimental import pallas as pl


def kernel(Z, R, edge_index, data, atom_emb, W_rbf_emb, W_emb, b_emb, Wo0_rbf, Wo0_d0, bo0_d0, Wo0_d1, bo0_d1, Wo0_d2, bo0_d2, Wo0_t, W1, b1, W2, b2, Wb, Wproj, Wskip, bskip, Wo1_rbf, Wo1_d0, bo1_d0, Wo1_d1, bo1_d1, Wo1_d2, bo1_d2, Wo1_t):
    raise NotImplementedError("write your pallas kernel here")


# speedup vs baseline: 8.1711x; 8.1711x over previous
import functools

import jax
import jax.numpy as jnp
import numpy as np
from jax import lax
from jax.experimental import pallas as pl
from jax.experimental.pallas import tpu as pltpu
from jax.experimental.pallas import tpu_sc as plsc

N = 50000
E = 800000
EMB = 32
NUM_RADIAL = 6
NUM_BILINEAR = 8
NUM_TARGETS = 12
CUTOFF = 5.0

NC = 2
NS = 16
NW = NC * NS
CHUNK = 128

NP = 50048
N_CHUNKS_NODE = NP // CHUNK
N_CHUNKS_EDGE = E // CHUNK
ROWS_PER_SUB = N // NS
ZCHUNK = 625
F32 = jnp.float32

EP = E // 4
NPAD = 50176
NPK = NPAD // 4
BEP = 2000
BNP = 1568


def _swish(x):
    return x * jax.nn.sigmoid(x)


def _mesh():
    return plsc.VectorSubcoreMesh(
        core_axis_name="c", subcore_axis_name="s", num_cores=NC, num_subcores=NS
    )


def _worker_id():
    return lax.axis_index("s") * NC + lax.axis_index("c")


def _sc_node_prep(z, ta, tb):
    @functools.partial(
        pl.kernel,
        out_type=(
            jax.ShapeDtypeStruct((NP, EMB), F32),
            jax.ShapeDtypeStruct((NP, EMB), F32),
        ),
        mesh=_mesh(),
        compiler_params=pltpu.CompilerParams(use_tc_tiling_on_sc=False),
        scratch_types=[
            pltpu.VMEM((CHUNK,), jnp.int32),
            pltpu.VMEM((CHUNK, EMB), F32),
            pltpu.VMEM((CHUNK, EMB), F32),
            pltpu.SemaphoreType.DMA,
        ],
    )
    def k(z_hbm, ta_hbm, tb_hbm, tan_hbm, tbn_hbm, ibuf, abuf, bbuf, sem):
        w = _worker_id()

        def body(j, carry):
            cid = j * NW + w

            @pl.when(cid < N_CHUNKS_NODE)
            def _():
                base = cid * CHUNK
                pltpu.sync_copy(z_hbm.at[pl.ds(base, CHUNK)], ibuf)
                pltpu.async_copy(ta_hbm.at[ibuf], abuf, sem).wait()
                pltpu.async_copy(tb_hbm.at[ibuf], bbuf, sem).wait()
                pltpu.sync_copy(abuf, tan_hbm.at[pl.ds(base, CHUNK)])
                pltpu.sync_copy(bbuf, tbn_hbm.at[pl.ds(base, CHUNK)])

            return carry

        lax.fori_loop(0, (N_CHUNKS_NODE + NW - 1) // NW, body, 0)

    return k(z, ta, tb)


def _sc_edge_gather(tan, tbn, r32, src, dst):
    @functools.partial(
        pl.kernel,
        out_type=(
            jax.ShapeDtypeStruct((E, EMB), F32),
            jax.ShapeDtypeStruct((E, EMB), F32),
            jax.ShapeDtypeStruct((E, EMB), F32),
        ),
        mesh=_mesh(),
        compiler_params=pltpu.CompilerParams(use_tc_tiling_on_sc=False),
        scratch_types=[
            pltpu.VMEM((CHUNK,), jnp.int32),
            pltpu.VMEM((CHUNK,), jnp.int32),
            pltpu.VMEM((CHUNK, EMB), F32),
            pltpu.VMEM((CHUNK, EMB), F32),
            pltpu.VMEM((CHUNK, EMB), F32),
            pltpu.VMEM((CHUNK, EMB), F32),
            pltpu.SemaphoreType.DMA,
        ],
    )
    def k(tan_hbm, tbn_hbm, r_hbm, src_hbm, dst_hbm, ab_hbm, rs_hbm, rd_hbm,
          isrc, idst, abuf, bbuf, rsbuf, rdbuf, sem):
        w = _worker_id()

        def body(j, carry):
            cid = j * NW + w

            @pl.when(cid < N_CHUNKS_EDGE)
            def _():
                base = cid * CHUNK
                pltpu.sync_copy(src_hbm.at[pl.ds(base, CHUNK)], isrc)
                pltpu.sync_copy(dst_hbm.at[pl.ds(base, CHUNK)], idst)
                pltpu.async_copy(tan_hbm.at[isrc], abuf, sem).wait()
                pltpu.async_copy(tbn_hbm.at[idst], bbuf, sem).wait()
                pltpu.async_copy(r_hbm.at[isrc], rsbuf, sem).wait()
                pltpu.async_copy(r_hbm.at[idst], rdbuf, sem).wait()

                def add_body(i, c2):
                    r = i * 4
                    for rr in range(4):
                        for h in range(0, EMB, 16):
                            sl = pl.ds(h, 16)
                            abuf[r + rr, sl] = abuf[r + rr, sl] + bbuf[r + rr, sl]
                    return c2

                lax.fori_loop(0, CHUNK // 4, add_body, 0)
                pltpu.sync_copy(abuf, ab_hbm.at[pl.ds(base, CHUNK)])
                pltpu.sync_copy(rsbuf, rs_hbm.at[pl.ds(base, CHUNK)])
                pltpu.sync_copy(rdbuf, rd_hbm.at[pl.ds(base, CHUNK)])

            return carry

        lax.fori_loop(0, (N_CHUNKS_EDGE + NW - 1) // NW, body, 0)

    return k(tan, tbn, r32, src, dst)


def _sc_scatter_add(upd, idx):
    @functools.partial(
        pl.kernel,
        out_type=jax.ShapeDtypeStruct((NC, NPAD, EMB), F32),
        mesh=_mesh(),
        compiler_params=pltpu.CompilerParams(use_tc_tiling_on_sc=False),
        scratch_types=[
            pltpu.VMEM_SHARED((N, EMB), F32),
            pltpu.VMEM((ZCHUNK, EMB), F32),
            pltpu.VMEM((CHUNK,), jnp.int32),
            pltpu.VMEM((CHUNK, EMB), F32),
        ],
    )
    def k(upd_hbm, idx_hbm, out_hbm, table, zbuf, ibuf, ubuf):
        c = lax.axis_index("c")
        s = lax.axis_index("s")
        w = s * NC + c
        zero16 = jnp.zeros((16,), F32)

        def zrow(i, carry):
            for h in range(0, EMB, 16):
                zbuf[i, pl.ds(h, 16)] = zero16
            return carry

        lax.fori_loop(0, ZCHUNK, zrow, 0)

        def zcopy(kk, carry):
            pltpu.sync_copy(
                zbuf, table.at[pl.ds(s * ROWS_PER_SUB + kk * ZCHUNK, ZCHUNK)]
            )
            return carry

        lax.fori_loop(0, ROWS_PER_SUB // ZCHUNK, zcopy, 0)
        plsc.subcore_barrier()

        def body(j, carry):
            cid = j * NW + w

            @pl.when(cid < N_CHUNKS_EDGE)
            def _():
                base = cid * CHUNK
                pltpu.sync_copy(idx_hbm.at[pl.ds(base, CHUNK)], ibuf)
                pltpu.sync_copy(upd_hbm.at[pl.ds(base, CHUNK)], ubuf)
                pltpu.sync_copy(ubuf, table.at[ibuf], add=True)

            return carry

        lax.fori_loop(0, (N_CHUNKS_EDGE + NW - 1) // NW, body, 0)
        plsc.subcore_barrier()

        def dump(kk, carry):
            r0 = s * ROWS_PER_SUB + kk * ZCHUNK
            pltpu.sync_copy(table.at[pl.ds(r0, ZCHUNK)], zbuf)
            pltpu.sync_copy(zbuf, out_hbm.at[c, pl.ds(r0, ZCHUNK)])
            return carry

        lax.fori_loop(0, ROWS_PER_SUB // ZCHUNK, dump, 0)

    return k(upd, idx)


def _sc_gather_pair(tlo, thi, src):
    @functools.partial(
        pl.kernel,
        out_type=(
            jax.ShapeDtypeStruct((E, EMB), F32),
            jax.ShapeDtypeStruct((E, EMB), F32),
        ),
        mesh=_mesh(),
        compiler_params=pltpu.CompilerParams(use_tc_tiling_on_sc=False),
        scratch_types=[
            pltpu.VMEM((CHUNK,), jnp.int32),
            pltpu.VMEM((CHUNK, EMB), F32),
            pltpu.VMEM((CHUNK, EMB), F32),
            pltpu.SemaphoreType.DMA,
        ],
    )
    def k(tlo_hbm, thi_hbm, src_hbm, olo_hbm, ohi_hbm, ibuf, lbuf, hbuf, sem):
        w = _worker_id()

        def body(j, carry):
            cid = j * NW + w

            @pl.when(cid < N_CHUNKS_EDGE)
            def _():
                base = cid * CHUNK
                pltpu.sync_copy(src_hbm.at[pl.ds(base, CHUNK)], ibuf)
                pltpu.async_copy(tlo_hbm.at[ibuf], lbuf, sem).wait()
                pltpu.async_copy(thi_hbm.at[ibuf], hbuf, sem).wait()
                pltpu.sync_copy(lbuf, olo_hbm.at[pl.ds(base, CHUNK)])
                pltpu.sync_copy(hbuf, ohi_hbm.at[pl.ds(base, CHUNK)])

            return carry

        lax.fori_loop(0, (N_CHUNKS_EDGE + NW - 1) // NW, body, 0)

    return k(tlo, thi, src)


_SIN_C = (3.1415926519453423, -5.167712606945147, 2.550161086578639,
          -0.5992457304355527, 0.08208905894537229, -0.007282179552874866,
          0.00039772714469602703)
_COS_C = (0.9999999999193145, -4.934802189550594, 4.05871188207033,
          -1.3352607090020963, 0.2353221275422643, -0.025787852309282922,
          0.0019059100562267845, -8.916912868632863e-05)


def _horner(t, coefs):
    r = jnp.full_like(t, coefs[-1])
    for c in coefs[-2::-1]:
        r = r * t + np.float32(c)
    return r


def _bd(w32):
    return jnp.kron(jnp.eye(4, dtype=F32), w32.astype(F32))


def _pad32(w, rows=None):
    w = jnp.asarray(w, F32)
    r = 32 if rows is None else rows
    out = jnp.zeros((32, 32), F32)
    return out.at[: w.shape[0], : w.shape[1]].set(w)


def _ucoef_table():
    u = np.zeros((8, 32), np.float32)
    rows = [[1], [0, 2], [-1, 0, 4], [0, -4, 0, 8], [1, 0, -12, 0, 16],
            [0, 6, 0, -32, 0, 32]]
    for k, cs in enumerate(rows):
        for j, c in enumerate(cs):
            u[j, k] = c
    return jnp.asarray(np.tile(u, (1, 4)))


def _lane_mask():
    lane = np.arange(128) % 32
    return jnp.asarray((lane < NUM_RADIAL).astype(np.float32)[None, :])


def _tile4(b):
    return jnp.tile(jnp.asarray(b, F32).reshape(1, -1), (1, 4))


def _tc_edge1(ab, rs, rd, bdj, bdwre, bdwo0r, bemb, ucoef, lmask):
    p = 5.0
    ca = -(p + 1.0) * (p + 2.0) / 2.0
    cb = p * (p + 2.0)
    cc = -p * (p + 1.0) / 2.0

    def body(ab_ref, rs_ref, rd_ref, bdj_ref, bdwre_ref, bdwo0r_ref,
             bemb_ref, u_ref, lm_ref, m_ref, g0_ref, rbf_ref):
        diff = rs_ref[...] - rd_ref[...]
        d2 = jnp.dot(diff * diff, bdj_ref[...], preferred_element_type=F32)
        d = jnp.sqrt(d2) + 1e-6
        x = d * (1.0 / CUTOFF)
        x2 = x * x
        x4 = x2 * x2
        env = 1.0 / x + ca * x4 + cb * x4 * x + cc * x4 * x2
        env = jnp.where(x < 1.0, env, 0.0)
        xc = jnp.minimum(x, 1.0)
        tt = xc * xc
        s1 = xc * _horner(tt, _SIN_C)
        c1 = _horner(tt, _COS_C)
        u = u_ref[...]
        U = jnp.broadcast_to(u[5:6, :], (BEP, 128))
        for j in range(4, -1, -1):
            U = U * c1 + u[j:j + 1, :]
        rbf = lm_ref[...] * (env * s1 * U)
        pre = ab_ref[...] + jnp.dot(rbf, bdwre_ref[...],
                                    preferred_element_type=F32) + bemb_ref[...]
        m = _swish(pre)
        g0 = jnp.dot(rbf, bdwo0r_ref[...], preferred_element_type=F32) * m
        m_ref[...] = m
        g0_ref[...] = g0
        rbf_ref[...] = rbf

    grid = EP // BEP
    eblk = pl.BlockSpec((BEP, 128), lambda i: (i, 0))
    wblk = pl.BlockSpec((128, 128), lambda i: (0, 0))
    rblk = pl.BlockSpec((1, 128), lambda i: (0, 0))
    return pl.pallas_call(
        body,
        grid=(grid,),
        in_specs=[eblk, eblk, eblk, wblk, wblk, wblk, rblk,
                  pl.BlockSpec((8, 128), lambda i: (0, 0)), rblk],
        out_specs=[eblk, eblk, eblk],
        out_shape=[
            jax.ShapeDtypeStruct((EP, 128), F32),
            jax.ShapeDtypeStruct((EP, 128), F32),
            jax.ShapeDtypeStruct((EP, 128), F32),
        ],
    )(ab, rs, rd, bdj, bdwre, bdwo0r, bemb, ucoef, lmask)


def _tc_node_mid(nap, t0p, bdw2, b2t, bdwblo, bdwbhi, bdwd, bdt, wtpk):
    def body(nap_ref, t0p_ref, bdw2_ref, b2_ref, wblo_ref, wbhi_ref,
             wd0_ref, bd0_ref, wd1_ref, bd1_ref, wd2_ref, bd2_ref, wt_ref,
             dlo_ref, dhi_ref, p0_ref):
        na = nap_ref[0] + nap_ref[1]
        c2 = _swish(jnp.dot(na, bdw2_ref[...], preferred_element_type=F32)
                    + b2_ref[...])
        dlo_ref[...] = jnp.dot(c2, wblo_ref[...], preferred_element_type=F32)
        dhi_ref[...] = jnp.dot(c2, wbhi_ref[...], preferred_element_type=F32)
        t = t0p_ref[0] + t0p_ref[1]
        t = _swish(jnp.dot(t, wd0_ref[...], preferred_element_type=F32) + bd0_ref[...])
        t = _swish(jnp.dot(t, wd1_ref[...], preferred_element_type=F32) + bd1_ref[...])
        t = _swish(jnp.dot(t, wd2_ref[...], preferred_element_type=F32) + bd2_ref[...])
        p0_ref[...] = jnp.dot(t, wt_ref[...], preferred_element_type=F32)

    grid = NPK // BNP
    nblk = pl.BlockSpec((BNP, 128), lambda i: (i, 0))
    pblk = pl.BlockSpec((NC, BNP, 128), lambda i: (0, i, 0))
    wblk = pl.BlockSpec((128, 128), lambda i: (0, 0))
    rblk = pl.BlockSpec((1, 128), lambda i: (0, 0))
    return pl.pallas_call(
        body,
        grid=(grid,),
        in_specs=[pblk, pblk, wblk, rblk, wblk, wblk,
                  wblk, rblk, wblk, rblk, wblk, rblk, wblk],
        out_specs=[nblk, nblk, nblk],
        out_shape=[
            jax.ShapeDtypeStruct((NPK, 128), F32),
            jax.ShapeDtypeStruct((NPK, 128), F32),
            jax.ShapeDtypeStruct((NPK, 128), F32),
        ],
    )(nap, t0p, bdw2, b2t, bdwblo, bdwbhi,
      bdwd[0], bdt[0], bdwd[1], bdt[1], bdwd[2], bdt[2], wtpk)


def _tc_edge2(m, rbf, dslo, dshi, bdw1, b1t, bdreplo, bdrephi, bdwplo,
              bdwphi, bdwskip, bskipt, bdwo1r):
    def body(m_ref, rbf_ref, dlo_ref, dhi_ref, w1_ref, b1_ref, rlo_ref,
             rhi_ref, plo_ref, phi_ref, wsk_ref, bsk_ref, wo1_ref, g1_ref):
        m = m_ref[...]
        rbf = rbf_ref[...]
        m_ji = _swish(jnp.dot(m, w1_ref[...], preferred_element_type=F32)
                      + b1_ref[...])
        xp = jnp.dot(jnp.dot(rbf, rlo_ref[...], preferred_element_type=F32)
                     * dlo_ref[...], plo_ref[...], preferred_element_type=F32)
        xp = xp + jnp.dot(jnp.dot(rbf, rhi_ref[...], preferred_element_type=F32)
                          * dhi_ref[...], phi_ref[...],
                          preferred_element_type=F32)
        m_new = _swish(jnp.dot(m_ji + xp, wsk_ref[...],
                               preferred_element_type=F32) + bsk_ref[...]) + m
        g1_ref[...] = jnp.dot(rbf, wo1_ref[...],
                              preferred_element_type=F32) * m_new

    grid = EP // BEP
    eblk = pl.BlockSpec((BEP, 128), lambda i: (i, 0))
    wblk = pl.BlockSpec((128, 128), lambda i: (0, 0))
    rblk = pl.BlockSpec((1, 128), lambda i: (0, 0))
    return pl.pallas_call(
        body,
        grid=(grid,),
        in_specs=[eblk, eblk, eblk, eblk, wblk, rblk, wblk, wblk, wblk,
                  wblk, wblk, rblk, wblk],
        out_specs=[eblk],
        out_shape=[jax.ShapeDtypeStruct((EP, 128), F32)],
    )(m, rbf, dslo, dshi, bdw1, b1t, bdreplo, bdrephi, bdwplo, bdwphi,
      bdwskip, bskipt, bdwo1r)[0]


def _tc_node_out(t1p, p0, bdwd, bdt, wtpk):
    def body(t1p_ref, p0_ref, wd0_ref, bd0_ref, wd1_ref, bd1_ref,
             wd2_ref, bd2_ref, wt_ref, p_ref):
        t = t1p_ref[0] + t1p_ref[1]
        t = _swish(jnp.dot(t, wd0_ref[...], preferred_element_type=F32) + bd0_ref[...])
        t = _swish(jnp.dot(t, wd1_ref[...], preferred_element_type=F32) + bd1_ref[...])
        t = _swish(jnp.dot(t, wd2_ref[...], preferred_element_type=F32) + bd2_ref[...])
        p_ref[...] = p0_ref[...] + jnp.dot(t, wt_ref[...],
                                           preferred_element_type=F32)

    grid = NPK // BNP
    nblk = pl.BlockSpec((BNP, 128), lambda i: (i, 0))
    pblk = pl.BlockSpec((NC, BNP, 128), lambda i: (0, i, 0))
    wblk = pl.BlockSpec((128, 128), lambda i: (0, 0))
    rblk = pl.BlockSpec((1, 128), lambda i: (0, 0))
    return pl.pallas_call(
        body,
        grid=(grid,),
        in_specs=[pblk, nblk, wblk, rblk, wblk, rblk, wblk, rblk, wblk],
        out_specs=[nblk],
        out_shape=[jax.ShapeDtypeStruct((NPK, 128), F32)],
    )(t1p, p0, bdwd[0], bdt[0], bdwd[1], bdt[1], bdwd[2], bdt[2], wtpk)[0]


def kernel(Z, R, edge_index, data, atom_emb, W_rbf_emb, W_emb, b_emb,
           Wo0_rbf, Wo0_d0, bo0_d0, Wo0_d1, bo0_d1, Wo0_d2, bo0_d2, Wo0_t,
           W1, b1, W2, b2, Wb, Wproj, Wskip, bskip,
           Wo1_rbf, Wo1_d0, bo1_d0, Wo1_d1, bo1_d1, Wo1_d2, bo1_d2, Wo1_t):
    src = edge_index[0].astype(jnp.int32)
    dst = edge_index[1].astype(jnp.int32)
    z32 = Z.astype(jnp.int32)

    ta = atom_emb @ W_emb[0:EMB]
    tb = atom_emb @ W_emb[EMB:2 * EMB]
    wre = W_rbf_emb @ W_emb[2 * EMB:3 * EMB]
    wbrs = jnp.transpose(Wb, (1, 0, 2)).reshape(EMB, NUM_RADIAL * NUM_BILINEAR)

    bdj = jnp.kron(jnp.eye(4, dtype=F32), jnp.ones((32, 32), F32))
    bdwre = _bd(_pad32(wre))
    bdwo0r = _bd(_pad32(Wo0_rbf))
    bdwo1r = _bd(_pad32(Wo1_rbf))
    bdw1 = _bd(W1)
    bdw2 = _bd(W2)
    bdwskip = _bd(Wskip)
    bdwblo = _bd(wbrs[:, 0:32])
    bdwbhi = _bd(_pad32(wbrs[:, 32:48]))

    replo = np.zeros((32, 32), np.float32)
    rephi = np.zeros((32, 32), np.float32)
    for r in range(4):
        for b in range(NUM_BILINEAR):
            replo[r, r * NUM_BILINEAR + b] = 1.0
    for r in range(4, 6):
        for b in range(NUM_BILINEAR):
            rephi[r, (r - 4) * NUM_BILINEAR + b] = 1.0
    bdreplo = _bd(jnp.asarray(replo))
    bdrephi = _bd(jnp.asarray(rephi))
    bdwplo = _bd(jnp.tile(Wproj, (4, 1)))
    bdwphi = _bd(_pad32(jnp.tile(Wproj, (2, 1))))

    bdwd0 = [_bd(Wo0_d0), _bd(Wo0_d1), _bd(Wo0_d2)]
    bdt0 = [_tile4(bo0_d0), _tile4(bo0_d1), _tile4(bo0_d2)]
    bdwd1 = [_bd(Wo1_d0), _bd(Wo1_d1), _bd(Wo1_d2)]
    bdt1 = [_tile4(bo1_d0), _tile4(bo1_d1), _tile4(bo1_d2)]
    wt0pk = _bd(_pad32(Wo0_t))
    wt1pk = _bd(_pad32(Wo1_t))

    zp = jnp.pad(z32, (0, NP - N))
    r32 = jnp.pad(R.astype(F32), ((0, NP - N), (0, EMB - 3)))

    tan, tbn = _sc_node_prep(zp, ta, tb)
    ab, rs, rd = _sc_edge_gather(tan, tbn, r32, src, dst)

    m, g0, rbf = _tc_edge1(ab.reshape(EP, 128), rs.reshape(EP, 128),
                           rd.reshape(EP, 128), bdj, bdwre, bdwo0r,
                           _tile4(b_emb), _ucoef_table(), _lane_mask())

    nap = _sc_scatter_add(m.reshape(E, EMB), dst)
    t0p = _sc_scatter_add(g0.reshape(E, EMB), dst)

    dlo, dhi, p0 = _tc_node_mid(nap.reshape(NC, NPK, 128),
                                t0p.reshape(NC, NPK, 128),
                                bdw2, _tile4(b2), bdwblo, bdwbhi,
                                bdwd0, bdt0, wt0pk)

    dslo, dshi = _sc_gather_pair(dlo.reshape(NPAD, EMB), dhi.reshape(NPAD, EMB), src)

    g1 = _tc_edge2(m, rbf, dslo.reshape(EP, 128), dshi.reshape(EP, 128),
                   bdw1, _tile4(b1), bdreplo, bdrephi, bdwplo, bdwphi,
                   bdwskip, _tile4(bskip), bdwo1r)

    t1p = _sc_scatter_add(g1.reshape(E, EMB), dst)

    ppk = _tc_node_out(t1p.reshape(NC, NPK, 128), p0, bdwd1, bdt1, wt1pk)
    return ppk.reshape(NPAD, EMB)[:N, :NUM_TARGETS]


# speedup vs baseline: 16.7446x; 2.0493x over previous
import functools

import jax
import jax.numpy as jnp
import numpy as np
from jax import lax
from jax.experimental import pallas as pl
from jax.experimental.pallas import tpu as pltpu
from jax.experimental.pallas import tpu_sc as plsc

N = 50000
E = 800000
EMB = 32
NUM_RADIAL = 6
NUM_BILINEAR = 8
NUM_TARGETS = 12
CUTOFF = 5.0

NC = 2
NS = 16
NW = NC * NS
CHUNK = 128

NP = 50048
N_CHUNKS_NODE = NP // CHUNK
N_CHUNKS_EDGE = E // CHUNK
ROWS_PER_SUB = N // NS
ZCHUNK = 625
F32 = jnp.float32

EP = E // 4
NPAD = 50176
NPK = NPAD // 4
BEP = 2000
BNP = 1568


def _swish(x):
    return x * jax.nn.sigmoid(x)


def _mesh():
    return plsc.VectorSubcoreMesh(
        core_axis_name="c", subcore_axis_name="s", num_cores=NC, num_subcores=NS
    )


def _worker_id():
    return lax.axis_index("s") * NC + lax.axis_index("c")


def _sc_node_prep(z, ta, tb):
    @functools.partial(
        pl.kernel,
        out_type=(
            jax.ShapeDtypeStruct((NP, EMB), F32),
            jax.ShapeDtypeStruct((NP, EMB), F32),
        ),
        mesh=_mesh(),
        compiler_params=pltpu.CompilerParams(use_tc_tiling_on_sc=False),
        scratch_types=[
            pltpu.VMEM((CHUNK,), jnp.int32),
            pltpu.VMEM((CHUNK, EMB), F32),
            pltpu.VMEM((CHUNK, EMB), F32),
            pltpu.SemaphoreType.DMA,
        ],
    )
    def k(z_hbm, ta_hbm, tb_hbm, tan_hbm, tbn_hbm, ibuf, abuf, bbuf, sem):
        w = _worker_id()

        def body(j, carry):
            cid = j * NW + w

            @pl.when(cid < N_CHUNKS_NODE)
            def _():
                base = cid * CHUNK
                pltpu.sync_copy(z_hbm.at[pl.ds(base, CHUNK)], ibuf)
                pltpu.async_copy(ta_hbm.at[ibuf], abuf, sem).wait()
                pltpu.async_copy(tb_hbm.at[ibuf], bbuf, sem).wait()
                pltpu.sync_copy(abuf, tan_hbm.at[pl.ds(base, CHUNK)])
                pltpu.sync_copy(bbuf, tbn_hbm.at[pl.ds(base, CHUNK)])

            return carry

        lax.fori_loop(0, (N_CHUNKS_NODE + NW - 1) // NW, body, 0)

    return k(z, ta, tb)


CPW_LO = N_CHUNKS_EDGE // NW
CPW_EXTRA = N_CHUNKS_EDGE - CPW_LO * NW
MAXC = CPW_LO + 1
NCH_PAD = MAXC * NW


def _worker_range():
    w = _worker_id()
    cnt = CPW_LO + (w < CPW_EXTRA).astype(jnp.int32)
    start = w * CPW_LO + jnp.minimum(w, CPW_EXTRA)
    return w, cnt, start


def _multi_gather(specs, tables, idx2ds):
    G = len(specs)
    num_idx = len(idx2ds)
    scratch = [pltpu.VMEM((MAXC, CHUNK), jnp.int32) for _ in range(num_idx)]
    scratch += [pltpu.VMEM((CHUNK, EMB), F32) for _ in range(2 * G)]
    scratch += [pltpu.SemaphoreType.DMA for _ in range(4)]

    @functools.partial(
        pl.kernel,
        out_type=tuple(jax.ShapeDtypeStruct((E, EMB), F32) for _ in range(G)),
        mesh=_mesh(),
        compiler_params=pltpu.CompilerParams(use_tc_tiling_on_sc=False),
        scratch_types=scratch,
    )
    def k(*refs):
        if True:
            tabs = refs[:G]
            idxs = refs[G:G + num_idx]
            outs = refs[G + num_idx:2 * G + num_idx]
            sc = refs[2 * G + num_idx:]
            idxv = sc[:num_idx]
            bufs = [[sc[num_idx + 2 * g + s] for s in range(2)] for g in range(G)]
            semg = [sc[num_idx + 2 * G], sc[num_idx + 2 * G + 1]]
            semw = [sc[num_idx + 2 * G + 2], sc[num_idx + 2 * G + 3]]

            w, cnt, start = _worker_range()
            for t in range(num_idx):
                pltpu.sync_copy(idxs[t].at[pl.ds(start, MAXC)], idxv[t])

            def wait_writes(j, slot):
                @pl.when((j >= 0) & (j < cnt))
                def _():
                    for g in range(G):
                        pltpu.make_async_copy(
                            bufs[g][slot], outs[g].at[pl.ds(0, CHUNK)],
                            semw[slot]).wait()

            def issue_gathers(j, slot):
                @pl.when(j < cnt)
                def _():
                    for g in range(G):
                        pltpu.async_copy(tabs[g].at[idxv[specs[g]].at[j]],
                                         bufs[g][slot], semg[slot])

            def finish_chunk(j, slot):
                @pl.when((j >= 0) & (j < cnt))
                def _():
                    for g in range(G):
                        pltpu.make_async_copy(
                            outs[g].at[pl.ds(0, CHUNK)],
                            bufs[g][slot], semg[slot]).wait()
                    base = (start + j) * CHUNK
                    for g in range(G):
                        pltpu.async_copy(bufs[g][slot],
                                         outs[g].at[pl.ds(base, CHUNK)],
                                         semw[slot])

            def body(i, carry):
                j0 = 2 * i
                j1 = 2 * i + 1
                wait_writes(j0 - 2, 0)
                issue_gathers(j0, 0)
                finish_chunk(j0 - 1, 1)
                wait_writes(j1 - 2, 1)
                issue_gathers(j1, 1)
                finish_chunk(j0, 0)
                return carry

            lax.fori_loop(0, (MAXC + 2 + 1) // 2, body, 0)

    return k(*tables, *idx2ds)


def _sc_edge_gather(tan, tbn, r32, src2d, dst2d):
    return _multi_gather((0, 1, 0, 1), (tan, tbn, r32, r32), (src2d, dst2d))


def _sc_gather_pair(tlo, thi, src2d):
    return _multi_gather((0, 0), (tlo, thi), (src2d,))


def _sc_scatter_add(upd, idx):
    @functools.partial(
        pl.kernel,
        out_type=jax.ShapeDtypeStruct((NC, NPAD, EMB), F32),
        mesh=_mesh(),
        compiler_params=pltpu.CompilerParams(use_tc_tiling_on_sc=False),
        scratch_types=[
            pltpu.VMEM_SHARED((N, EMB), F32),
            pltpu.VMEM((ZCHUNK, EMB), F32),
            pltpu.VMEM((CHUNK,), jnp.int32),
            pltpu.VMEM((CHUNK,), jnp.int32),
            pltpu.VMEM((CHUNK, EMB), F32),
            pltpu.VMEM((CHUNK, EMB), F32),
            pltpu.SemaphoreType.DMA,
            pltpu.SemaphoreType.DMA,
        ],
    )
    def k(upd_hbm, idx_hbm, out_hbm, table, zbuf, ib0, ib1, ub0, ub1,
          sem0, sem1):
        c = lax.axis_index("c")
        s = lax.axis_index("s")
        w, cnt, start = _worker_range()
        ibufs = [ib0, ib1]
        ubufs = [ub0, ub1]
        sems = [sem0, sem1]
        zero16 = jnp.zeros((16,), F32)

        def zrow(i, carry):
            for h in range(0, EMB, 16):
                zbuf[i, pl.ds(h, 16)] = zero16
            return carry

        lax.fori_loop(0, ZCHUNK, zrow, 0)

        def zcopy(kk, carry):
            pltpu.sync_copy(
                zbuf, table.at[pl.ds(s * ROWS_PER_SUB + kk * ZCHUNK, ZCHUNK)]
            )
            return carry

        lax.fori_loop(0, ROWS_PER_SUB // ZCHUNK, zcopy, 0)
        plsc.subcore_barrier()

        def issue(j, slot):
            @pl.when(j < cnt)
            def _():
                base = (start + j) * CHUNK
                pltpu.async_copy(idx_hbm.at[pl.ds(base, CHUNK)], ibufs[slot],
                                 sems[slot])
                pltpu.async_copy(upd_hbm.at[pl.ds(base, CHUNK)], ubufs[slot],
                                 sems[slot])

        def process(j, slot):
            @pl.when((j >= 0) & (j < cnt))
            def _():
                pltpu.make_async_copy(idx_hbm.at[pl.ds(0, CHUNK)],
                                      ibufs[slot], sems[slot]).wait()
                pltpu.make_async_copy(upd_hbm.at[pl.ds(0, CHUNK)],
                                      ubufs[slot], sems[slot]).wait()
                pltpu.sync_copy(ubufs[slot], table.at[ibufs[slot]], add=True)

        def body(i, carry):
            j0 = 2 * i
            j1 = 2 * i + 1
            issue(j0, 0)
            process(j0 - 1, 1)
            issue(j1, 1)
            process(j0, 0)
            return carry

        lax.fori_loop(0, (MAXC + 2) // 2, body, 0)
        plsc.subcore_barrier()

        def dump(kk, carry):
            r0 = s * ROWS_PER_SUB + kk * ZCHUNK
            pltpu.sync_copy(table.at[pl.ds(r0, ZCHUNK)], zbuf)
            pltpu.sync_copy(zbuf, out_hbm.at[c, pl.ds(r0, ZCHUNK)])
            return carry

        lax.fori_loop(0, ROWS_PER_SUB // ZCHUNK, dump, 0)

    return k(upd, idx)


_SIN_C = (3.1415926519453423, -5.167712606945147, 2.550161086578639,
          -0.5992457304355527, 0.08208905894537229, -0.007282179552874866,
          0.00039772714469602703)
_COS_C = (0.9999999999193145, -4.934802189550594, 4.05871188207033,
          -1.3352607090020963, 0.2353221275422643, -0.025787852309282922,
          0.0019059100562267845, -8.916912868632863e-05)


def _horner(t, coefs):
    r = jnp.full_like(t, coefs[-1])
    for c in coefs[-2::-1]:
        r = r * t + np.float32(c)
    return r


def _bd(w32):
    return jnp.kron(jnp.eye(4, dtype=F32), w32.astype(F32))


def _pad32(w, rows=None):
    w = jnp.asarray(w, F32)
    r = 32 if rows is None else rows
    out = jnp.zeros((32, 32), F32)
    return out.at[: w.shape[0], : w.shape[1]].set(w)


def _ucoef_table():
    u = np.zeros((8, 32), np.float32)
    rows = [[1], [0, 2], [-1, 0, 4], [0, -4, 0, 8], [1, 0, -12, 0, 16],
            [0, 6, 0, -32, 0, 32]]
    for k, cs in enumerate(rows):
        for j, c in enumerate(cs):
            u[j, k] = c
    return jnp.asarray(np.tile(u, (1, 4)))


def _lane_mask():
    lane = np.arange(128) % 32
    return jnp.asarray((lane < NUM_RADIAL).astype(np.float32)[None, :])


def _tile4(b):
    return jnp.tile(jnp.asarray(b, F32).reshape(1, -1), (1, 4))


def _tc_edge1(an, bn, rs, rd, bdj, bdwre, bdwo0r, bemb, ucoef, lmask):
    p = 5.0
    ca = -(p + 1.0) * (p + 2.0) / 2.0
    cb = p * (p + 2.0)
    cc = -p * (p + 1.0) / 2.0

    def body(an_ref, bn_ref, rs_ref, rd_ref, bdj_ref, bdwre_ref, bdwo0r_ref,
             bemb_ref, u_ref, lm_ref, m_ref, g0_ref, rbf_ref):
        diff = rs_ref[...] - rd_ref[...]
        d2 = jnp.dot(diff * diff, bdj_ref[...], preferred_element_type=F32)
        d = jnp.sqrt(d2) + 1e-6
        x = d * (1.0 / CUTOFF)
        x2 = x * x
        x4 = x2 * x2
        env = 1.0 / x + ca * x4 + cb * x4 * x + cc * x4 * x2
        env = jnp.where(x < 1.0, env, 0.0)
        xc = jnp.minimum(x, 1.0)
        tt = xc * xc
        s1 = xc * _horner(tt, _SIN_C)
        c1 = _horner(tt, _COS_C)
        u = u_ref[...]
        U = jnp.broadcast_to(u[5:6, :], (BEP, 128))
        for j in range(4, -1, -1):
            U = U * c1 + u[j:j + 1, :]
        rbf = lm_ref[...] * (env * s1 * U)
        pre = an_ref[...] + bn_ref[...] + jnp.dot(
            rbf, bdwre_ref[...], preferred_element_type=F32) + bemb_ref[...]
        m = _swish(pre)
        g0 = jnp.dot(rbf, bdwo0r_ref[...], preferred_element_type=F32) * m
        m_ref[...] = m
        g0_ref[...] = g0
        rbf_ref[...] = rbf

    grid = EP // BEP
    eblk = pl.BlockSpec((BEP, 128), lambda i: (i, 0))
    wblk = pl.BlockSpec((128, 128), lambda i: (0, 0))
    rblk = pl.BlockSpec((1, 128), lambda i: (0, 0))
    return pl.pallas_call(
        body,
        grid=(grid,),
        in_specs=[eblk, eblk, eblk, eblk, wblk, wblk, wblk, rblk,
                  pl.BlockSpec((8, 128), lambda i: (0, 0)), rblk],
        out_specs=[eblk, eblk, eblk],
        out_shape=[
            jax.ShapeDtypeStruct((EP, 128), F32),
            jax.ShapeDtypeStruct((EP, 128), F32),
            jax.ShapeDtypeStruct((EP, 128), F32),
        ],
    )(an, bn, rs, rd, bdj, bdwre, bdwo0r, bemb, ucoef, lmask)


def _tc_node_mid(nap, t0p, bdw2, b2t, bdwblo, bdwbhi, bdwd, bdt, wtpk):
    def body(nap_ref, t0p_ref, bdw2_ref, b2_ref, wblo_ref, wbhi_ref,
             wd0_ref, bd0_ref, wd1_ref, bd1_ref, wd2_ref, bd2_ref, wt_ref,
             dlo_ref, dhi_ref, p0_ref):
        na = nap_ref[0] + nap_ref[1]
        c2 = _swish(jnp.dot(na, bdw2_ref[...], preferred_element_type=F32)
                    + b2_ref[...])
        dlo_ref[...] = jnp.dot(c2, wblo_ref[...], preferred_element_type=F32)
        dhi_ref[...] = jnp.dot(c2, wbhi_ref[...], preferred_element_type=F32)
        t = t0p_ref[0] + t0p_ref[1]
        t = _swish(jnp.dot(t, wd0_ref[...], preferred_element_type=F32) + bd0_ref[...])
        t = _swish(jnp.dot(t, wd1_ref[...], preferred_element_type=F32) + bd1_ref[...])
        t = _swish(jnp.dot(t, wd2_ref[...], preferred_element_type=F32) + bd2_ref[...])
        p0_ref[...] = jnp.dot(t, wt_ref[...], preferred_element_type=F32)

    grid = NPK // BNP
    nblk = pl.BlockSpec((BNP, 128), lambda i: (i, 0))
    pblk = pl.BlockSpec((NC, BNP, 128), lambda i: (0, i, 0))
    wblk = pl.BlockSpec((128, 128), lambda i: (0, 0))
    rblk = pl.BlockSpec((1, 128), lambda i: (0, 0))
    return pl.pallas_call(
        body,
        grid=(grid,),
        in_specs=[pblk, pblk, wblk, rblk, wblk, wblk,
                  wblk, rblk, wblk, rblk, wblk, rblk, wblk],
        out_specs=[nblk, nblk, nblk],
        out_shape=[
            jax.ShapeDtypeStruct((NPK, 128), F32),
            jax.ShapeDtypeStruct((NPK, 128), F32),
            jax.ShapeDtypeStruct((NPK, 128), F32),
        ],
    )(nap, t0p, bdw2, b2t, bdwblo, bdwbhi,
      bdwd[0], bdt[0], bdwd[1], bdt[1], bdwd[2], bdt[2], wtpk)


def _tc_edge2(m, rbf, dslo, dshi, bdw1, b1t, bdreplo, bdrephi, bdwplo,
              bdwphi, bdwskip, bskipt, bdwo1r):
    def body(m_ref, rbf_ref, dlo_ref, dhi_ref, w1_ref, b1_ref, rlo_ref,
             rhi_ref, plo_ref, phi_ref, wsk_ref, bsk_ref, wo1_ref, g1_ref):
        m = m_ref[...]
        rbf = rbf_ref[...]
        m_ji = _swish(jnp.dot(m, w1_ref[...], preferred_element_type=F32)
                      + b1_ref[...])
        xp = jnp.dot(jnp.dot(rbf, rlo_ref[...], preferred_element_type=F32)
                     * dlo_ref[...], plo_ref[...], preferred_element_type=F32)
        xp = xp + jnp.dot(jnp.dot(rbf, rhi_ref[...], preferred_element_type=F32)
                          * dhi_ref[...], phi_ref[...],
                          preferred_element_type=F32)
        m_new = _swish(jnp.dot(m_ji + xp, wsk_ref[...],
                               preferred_element_type=F32) + bsk_ref[...]) + m
        g1_ref[...] = jnp.dot(rbf, wo1_ref[...],
                              preferred_element_type=F32) * m_new

    grid = EP // BEP
    eblk = pl.BlockSpec((BEP, 128), lambda i: (i, 0))
    wblk = pl.BlockSpec((128, 128), lambda i: (0, 0))
    rblk = pl.BlockSpec((1, 128), lambda i: (0, 0))
    return pl.pallas_call(
        body,
        grid=(grid,),
        in_specs=[eblk, eblk, eblk, eblk, wblk, rblk, wblk, wblk, wblk,
                  wblk, wblk, rblk, wblk],
        out_specs=[eblk],
        out_shape=[jax.ShapeDtypeStruct((EP, 128), F32)],
    )(m, rbf, dslo, dshi, bdw1, b1t, bdreplo, bdrephi, bdwplo, bdwphi,
      bdwskip, bskipt, bdwo1r)[0]


def _tc_node_out(t1p, p0, bdwd, bdt, wtpk):
    def body(t1p_ref, p0_ref, wd0_ref, bd0_ref, wd1_ref, bd1_ref,
             wd2_ref, bd2_ref, wt_ref, p_ref):
        t = t1p_ref[0] + t1p_ref[1]
        t = _swish(jnp.dot(t, wd0_ref[...], preferred_element_type=F32) + bd0_ref[...])
        t = _swish(jnp.dot(t, wd1_ref[...], preferred_element_type=F32) + bd1_ref[...])
        t = _swish(jnp.dot(t, wd2_ref[...], preferred_element_type=F32) + bd2_ref[...])
        p_ref[...] = p0_ref[...] + jnp.dot(t, wt_ref[...],
                                           preferred_element_type=F32)

    grid = NPK // BNP
    nblk = pl.BlockSpec((BNP, 128), lambda i: (i, 0))
    pblk = pl.BlockSpec((NC, BNP, 128), lambda i: (0, i, 0))
    wblk = pl.BlockSpec((128, 128), lambda i: (0, 0))
    rblk = pl.BlockSpec((1, 128), lambda i: (0, 0))
    return pl.pallas_call(
        body,
        grid=(grid,),
        in_specs=[pblk, nblk, wblk, rblk, wblk, rblk, wblk, rblk, wblk],
        out_specs=[nblk],
        out_shape=[jax.ShapeDtypeStruct((NPK, 128), F32)],
    )(t1p, p0, bdwd[0], bdt[0], bdwd[1], bdt[1], bdwd[2], bdt[2], wtpk)[0]


def kernel(Z, R, edge_index, data, atom_emb, W_rbf_emb, W_emb, b_emb,
           Wo0_rbf, Wo0_d0, bo0_d0, Wo0_d1, bo0_d1, Wo0_d2, bo0_d2, Wo0_t,
           W1, b1, W2, b2, Wb, Wproj, Wskip, bskip,
           Wo1_rbf, Wo1_d0, bo1_d0, Wo1_d1, bo1_d1, Wo1_d2, bo1_d2, Wo1_t):
    src = edge_index[0].astype(jnp.int32)
    dst = edge_index[1].astype(jnp.int32)
    z32 = Z.astype(jnp.int32)

    ta = atom_emb @ W_emb[0:EMB]
    tb = atom_emb @ W_emb[EMB:2 * EMB]
    wre = W_rbf_emb @ W_emb[2 * EMB:3 * EMB]
    wbrs = jnp.transpose(Wb, (1, 0, 2)).reshape(EMB, NUM_RADIAL * NUM_BILINEAR)

    bdj = jnp.kron(jnp.eye(4, dtype=F32), jnp.ones((32, 32), F32))
    bdwre = _bd(_pad32(wre))
    bdwo0r = _bd(_pad32(Wo0_rbf))
    bdwo1r = _bd(_pad32(Wo1_rbf))
    bdw1 = _bd(W1)
    bdw2 = _bd(W2)
    bdwskip = _bd(Wskip)
    bdwblo = _bd(wbrs[:, 0:32])
    bdwbhi = _bd(_pad32(wbrs[:, 32:48]))

    replo = np.zeros((32, 32), np.float32)
    rephi = np.zeros((32, 32), np.float32)
    for r in range(4):
        for b in range(NUM_BILINEAR):
            replo[r, r * NUM_BILINEAR + b] = 1.0
    for r in range(4, 6):
        for b in range(NUM_BILINEAR):
            rephi[r, (r - 4) * NUM_BILINEAR + b] = 1.0
    bdreplo = _bd(jnp.asarray(replo))
    bdrephi = _bd(jnp.asarray(rephi))
    bdwplo = _bd(jnp.tile(Wproj, (4, 1)))
    bdwphi = _bd(_pad32(jnp.tile(Wproj, (2, 1))))

    bdwd0 = [_bd(Wo0_d0), _bd(Wo0_d1), _bd(Wo0_d2)]
    bdt0 = [_tile4(bo0_d0), _tile4(bo0_d1), _tile4(bo0_d2)]
    bdwd1 = [_bd(Wo1_d0), _bd(Wo1_d1), _bd(Wo1_d2)]
    bdt1 = [_tile4(bo1_d0), _tile4(bo1_d1), _tile4(bo1_d2)]
    wt0pk = _bd(_pad32(Wo0_t))
    wt1pk = _bd(_pad32(Wo1_t))

    zp = jnp.pad(z32, (0, NP - N))
    r32 = jnp.pad(R.astype(F32), ((0, NP - N), (0, EMB - 3)))

    src2d = jnp.pad(src.reshape(N_CHUNKS_EDGE, CHUNK),
                    ((0, NCH_PAD - N_CHUNKS_EDGE), (0, 0)))
    dst2d = jnp.pad(dst.reshape(N_CHUNKS_EDGE, CHUNK),
                    ((0, NCH_PAD - N_CHUNKS_EDGE), (0, 0)))

    tan, tbn = _sc_node_prep(zp, ta, tb)
    an, bn, rs, rd = _sc_edge_gather(tan, tbn, r32, src2d, dst2d)

    m, g0, rbf = _tc_edge1(an.reshape(EP, 128), bn.reshape(EP, 128),
                           rs.reshape(EP, 128), rd.reshape(EP, 128),
                           bdj, bdwre, bdwo0r,
                           _tile4(b_emb), _ucoef_table(), _lane_mask())

    nap = _sc_scatter_add(m.reshape(E, EMB), dst)
    t0p = _sc_scatter_add(g0.reshape(E, EMB), dst)

    dlo, dhi, p0 = _tc_node_mid(nap.reshape(NC, NPK, 128),
                                t0p.reshape(NC, NPK, 128),
                                bdw2, _tile4(b2), bdwblo, bdwbhi,
                                bdwd0, bdt0, wt0pk)

    dslo, dshi = _sc_gather_pair(dlo.reshape(NPAD, EMB), dhi.reshape(NPAD, EMB),
                                 src2d)

    g1 = _tc_edge2(m, rbf, dslo.reshape(EP, 128), dshi.reshape(EP, 128),
                   bdw1, _tile4(b1), bdreplo, bdrephi, bdwplo, bdwphi,
                   bdwskip, _tile4(bskip), bdwo1r)

    t1p = _sc_scatter_add(g1.reshape(E, EMB), dst)

    ppk = _tc_node_out(t1p.reshape(NC, NPK, 128), p0, bdwd1, bdt1, wt1pk)
    return ppk.reshape(NPAD, EMB)[:N, :NUM_TARGETS]


# speedup vs baseline: 17.1643x; 1.0251x over previous
import functools

import jax
import jax.numpy as jnp
import numpy as np
from jax import lax
from jax.experimental import pallas as pl
from jax.experimental.pallas import tpu as pltpu
from jax.experimental.pallas import tpu_sc as plsc

N = 50000
E = 800000
EMB = 32
NUM_RADIAL = 6
NUM_BILINEAR = 8
NUM_TARGETS = 12
CUTOFF = 5.0

NC = 2
NS = 16
NW = NC * NS
CHUNK = 128

NP = 50048
N_CHUNKS_NODE = NP // CHUNK
N_CHUNKS_EDGE = E // CHUNK
ROWS_PER_SUB = N // NS
ZCHUNK = 625
F32 = jnp.float32

EP = E // 4
NPAD = 50176
NPK = NPAD // 4
BEP = 2000
BNP = 1568


def _swish(x):
    return x * jax.nn.sigmoid(x)


def _mesh():
    return plsc.VectorSubcoreMesh(
        core_axis_name="c", subcore_axis_name="s", num_cores=NC, num_subcores=NS
    )


def _worker_id():
    return lax.axis_index("s") * NC + lax.axis_index("c")


def _sc_node_prep(z, ta, tb):
    @functools.partial(
        pl.kernel,
        out_type=(
            jax.ShapeDtypeStruct((NP, EMB), F32),
            jax.ShapeDtypeStruct((NP, EMB), F32),
        ),
        mesh=_mesh(),
        compiler_params=pltpu.CompilerParams(use_tc_tiling_on_sc=False),
        scratch_types=[
            pltpu.VMEM((CHUNK,), jnp.int32),
            pltpu.VMEM((CHUNK, EMB), F32),
            pltpu.VMEM((CHUNK, EMB), F32),
            pltpu.SemaphoreType.DMA,
        ],
    )
    def k(z_hbm, ta_hbm, tb_hbm, tan_hbm, tbn_hbm, ibuf, abuf, bbuf, sem):
        w = _worker_id()

        def body(j, carry):
            cid = j * NW + w

            @pl.when(cid < N_CHUNKS_NODE)
            def _():
                base = cid * CHUNK
                pltpu.sync_copy(z_hbm.at[pl.ds(base, CHUNK)], ibuf)
                pltpu.async_copy(ta_hbm.at[ibuf], abuf, sem).wait()
                pltpu.async_copy(tb_hbm.at[ibuf], bbuf, sem).wait()
                pltpu.sync_copy(abuf, tan_hbm.at[pl.ds(base, CHUNK)])
                pltpu.sync_copy(bbuf, tbn_hbm.at[pl.ds(base, CHUNK)])

            return carry

        lax.fori_loop(0, (N_CHUNKS_NODE + NW - 1) // NW, body, 0)

    return k(z, ta, tb)


CPW_LO = N_CHUNKS_EDGE // NW
CPW_EXTRA = N_CHUNKS_EDGE - CPW_LO * NW
MAXC = CPW_LO + 1
NCH_PAD = MAXC * NW


def _worker_range():
    w = _worker_id()
    cnt = CPW_LO + (w < CPW_EXTRA).astype(jnp.int32)
    start = w * CPW_LO + jnp.minimum(w, CPW_EXTRA)
    return w, cnt, start


def _multi_gather(specs, tables, idx2ds, fuse_add=False):
    G = len(specs)
    GO = G - 1 if fuse_add else G
    num_idx = len(idx2ds)
    scratch = [pltpu.VMEM((MAXC, CHUNK), jnp.int32) for _ in range(num_idx)]
    scratch += [pltpu.VMEM((CHUNK, EMB), F32) for _ in range(2 * G)]
    scratch += [pltpu.SemaphoreType.DMA for _ in range(4)]

    @functools.partial(
        pl.kernel,
        out_type=tuple(jax.ShapeDtypeStruct((E, EMB), F32) for _ in range(GO)),
        mesh=_mesh(),
        compiler_params=pltpu.CompilerParams(use_tc_tiling_on_sc=False),
        scratch_types=scratch,
    )
    def k(*refs):
        if True:
            tabs = refs[:G]
            idxs = refs[G:G + num_idx]
            outs = refs[G + num_idx:G + GO + num_idx]
            sc = refs[G + GO + num_idx:]
            idxv = sc[:num_idx]
            bufs = [[sc[num_idx + 2 * g + s] for s in range(2)] for g in range(G)]
            semg = [sc[num_idx + 2 * G], sc[num_idx + 2 * G + 1]]
            semw = [sc[num_idx + 2 * G + 2], sc[num_idx + 2 * G + 3]]

            w, cnt, start = _worker_range()
            for t in range(num_idx):
                pltpu.sync_copy(idxs[t].at[pl.ds(start, MAXC)], idxv[t])

            def wait_writes(j, slot):
                @pl.when((j >= 0) & (j < cnt))
                def _():
                    for g in range(GO):
                        pltpu.make_async_copy(
                            bufs[g][slot], outs[g].at[pl.ds(0, CHUNK)],
                            semw[slot]).wait()

            def issue_gathers(j, slot):
                @pl.when(j < cnt)
                def _():
                    for g in range(G):
                        pltpu.async_copy(tabs[g].at[idxv[specs[g]].at[j]],
                                         bufs[g][slot], semg[slot])

            def finish_chunk(j, slot):
                @pl.when((j >= 0) & (j < cnt))
                def _():
                    for g in range(G):
                        pltpu.make_async_copy(
                            outs[0].at[pl.ds(0, CHUNK)],
                            bufs[g][slot], semg[slot]).wait()
                    if fuse_add:
                        a = bufs[0][slot]
                        b = bufs[1][slot]

                        def add_body(i, c2):
                            r = i * 4
                            for rr in range(4):
                                for h in range(0, EMB, 16):
                                    sl = pl.ds(h, 16)
                                    a[r + rr, sl] = a[r + rr, sl] + b[r + rr, sl]
                            return c2

                        lax.fori_loop(0, CHUNK // 4, add_body, 0)
                    base = (start + j) * CHUNK
                    wbufs = ([bufs[0]] + bufs[2:]) if fuse_add else bufs
                    for g in range(GO):
                        pltpu.async_copy(wbufs[g][slot],
                                         outs[g].at[pl.ds(base, CHUNK)],
                                         semw[slot])

            def body(i, carry):
                j0 = 2 * i
                j1 = 2 * i + 1
                wait_writes(j0 - 2, 0)
                issue_gathers(j0, 0)
                finish_chunk(j0 - 1, 1)
                wait_writes(j1 - 2, 1)
                issue_gathers(j1, 1)
                finish_chunk(j0, 0)
                return carry

            lax.fori_loop(0, (MAXC + 2 + 1) // 2, body, 0)

    return k(*tables, *idx2ds)


def _sc_edge_gather(tan, tbn, r32, src2d, dst2d):
    return _multi_gather((0, 1, 0, 1), (tan, tbn, r32, r32), (src2d, dst2d),
                         fuse_add=True)


def _sc_gather_pair(tlo, thi, src2d):
    return _multi_gather((0, 0), (tlo, thi), (src2d,))


def _sc_scatter_add(upd, idx):
    @functools.partial(
        pl.kernel,
        out_type=jax.ShapeDtypeStruct((NC, NPAD, EMB), F32),
        mesh=_mesh(),
        compiler_params=pltpu.CompilerParams(use_tc_tiling_on_sc=False),
        scratch_types=[
            pltpu.VMEM_SHARED((N, EMB), F32),
            pltpu.VMEM((ZCHUNK, EMB), F32),
            pltpu.VMEM((CHUNK,), jnp.int32),
            pltpu.VMEM((CHUNK,), jnp.int32),
            pltpu.VMEM((CHUNK, EMB), F32),
            pltpu.VMEM((CHUNK, EMB), F32),
            pltpu.SemaphoreType.DMA,
            pltpu.SemaphoreType.DMA,
        ],
    )
    def k(upd_hbm, idx_hbm, out_hbm, table, zbuf, ib0, ib1, ub0, ub1,
          sem0, sem1):
        c = lax.axis_index("c")
        s = lax.axis_index("s")
        w, cnt, start = _worker_range()
        ibufs = [ib0, ib1]
        ubufs = [ub0, ub1]
        sems = [sem0, sem1]
        zero16 = jnp.zeros((16,), F32)

        def zrow(i, carry):
            for h in range(0, EMB, 16):
                zbuf[i, pl.ds(h, 16)] = zero16
            return carry

        lax.fori_loop(0, ZCHUNK, zrow, 0)

        def zcopy(kk, carry):
            pltpu.sync_copy(
                zbuf, table.at[pl.ds(s * ROWS_PER_SUB + kk * ZCHUNK, ZCHUNK)]
            )
            return carry

        lax.fori_loop(0, ROWS_PER_SUB // ZCHUNK, zcopy, 0)
        plsc.subcore_barrier()

        def issue(j, slot):
            @pl.when(j < cnt)
            def _():
                base = (start + j) * CHUNK
                pltpu.async_copy(idx_hbm.at[pl.ds(base, CHUNK)], ibufs[slot],
                                 sems[slot])
                pltpu.async_copy(upd_hbm.at[pl.ds(base, CHUNK)], ubufs[slot],
                                 sems[slot])

        def process(j, slot):
            @pl.when((j >= 0) & (j < cnt))
            def _():
                pltpu.make_async_copy(idx_hbm.at[pl.ds(0, CHUNK)],
                                      ibufs[slot], sems[slot]).wait()
                pltpu.make_async_copy(upd_hbm.at[pl.ds(0, CHUNK)],
                                      ubufs[slot], sems[slot]).wait()
                pltpu.sync_copy(ubufs[slot], table.at[ibufs[slot]], add=True)

        def body(i, carry):
            j0 = 2 * i
            j1 = 2 * i + 1
            issue(j0, 0)
            process(j0 - 1, 1)
            issue(j1, 1)
            process(j0, 0)
            return carry

        lax.fori_loop(0, (MAXC + 2) // 2, body, 0)
        plsc.subcore_barrier()

        def dump(kk, carry):
            r0 = s * ROWS_PER_SUB + kk * ZCHUNK
            pltpu.sync_copy(table.at[pl.ds(r0, ZCHUNK)], zbuf)
            pltpu.sync_copy(zbuf, out_hbm.at[c, pl.ds(r0, ZCHUNK)])
            return carry

        lax.fori_loop(0, ROWS_PER_SUB // ZCHUNK, dump, 0)

    return k(upd, idx)


_SIN_C = (3.1415926519453423, -5.167712606945147, 2.550161086578639,
          -0.5992457304355527, 0.08208905894537229, -0.007282179552874866,
          0.00039772714469602703)
_COS_C = (0.9999999999193145, -4.934802189550594, 4.05871188207033,
          -1.3352607090020963, 0.2353221275422643, -0.025787852309282922,
          0.0019059100562267845, -8.916912868632863e-05)


def _horner(t, coefs):
    r = jnp.full_like(t, coefs[-1])
    for c in coefs[-2::-1]:
        r = r * t + np.float32(c)
    return r


def _bd(w32):
    return jnp.kron(jnp.eye(4, dtype=F32), w32.astype(F32))


def _pad32(w, rows=None):
    w = jnp.asarray(w, F32)
    r = 32 if rows is None else rows
    out = jnp.zeros((32, 32), F32)
    return out.at[: w.shape[0], : w.shape[1]].set(w)


def _ucoef_table():
    u = np.zeros((8, 32), np.float32)
    rows = [[1], [0, 2], [-1, 0, 4], [0, -4, 0, 8], [1, 0, -12, 0, 16],
            [0, 6, 0, -32, 0, 32]]
    for k, cs in enumerate(rows):
        for j, c in enumerate(cs):
            u[j, k] = c
    return jnp.asarray(np.tile(u, (1, 4)))


def _lane_mask():
    lane = np.arange(128) % 32
    return jnp.asarray((lane < NUM_RADIAL).astype(np.float32)[None, :])


def _tile4(b):
    return jnp.tile(jnp.asarray(b, F32).reshape(1, -1), (1, 4))


def _tc_edge1(ab, rs, rd, bdj, bdwre, bdwo0r, bemb, ucoef, lmask):
    p = 5.0
    ca = -(p + 1.0) * (p + 2.0) / 2.0
    cb = p * (p + 2.0)
    cc = -p * (p + 1.0) / 2.0

    def body(ab_ref, rs_ref, rd_ref, bdj_ref, bdwre_ref, bdwo0r_ref,
             bemb_ref, u_ref, lm_ref, m_ref, g0_ref, rbf_ref):
        diff = rs_ref[...] - rd_ref[...]
        d2 = jnp.dot(diff * diff, bdj_ref[...], preferred_element_type=F32)
        d = jnp.sqrt(d2) + 1e-6
        x = d * (1.0 / CUTOFF)
        x2 = x * x
        x4 = x2 * x2
        env = 1.0 / x + ca * x4 + cb * x4 * x + cc * x4 * x2
        env = jnp.where(x < 1.0, env, 0.0)
        xc = jnp.minimum(x, 1.0)
        tt = xc * xc
        s1 = xc * _horner(tt, _SIN_C)
        c1 = _horner(tt, _COS_C)
        u = u_ref[...]
        U = jnp.broadcast_to(u[5:6, :], (BEP, 128))
        for j in range(4, -1, -1):
            U = U * c1 + u[j:j + 1, :]
        rbf = lm_ref[...] * (env * s1 * U)
        pre = ab_ref[...] + jnp.dot(
            rbf, bdwre_ref[...], preferred_element_type=F32) + bemb_ref[...]
        m = _swish(pre)
        g0 = jnp.dot(rbf, bdwo0r_ref[...], preferred_element_type=F32) * m
        m_ref[...] = m
        g0_ref[...] = g0
        rbf_ref[...] = rbf

    grid = EP // BEP
    eblk = pl.BlockSpec((BEP, 128), lambda i: (i, 0))
    wblk = pl.BlockSpec((128, 128), lambda i: (0, 0))
    rblk = pl.BlockSpec((1, 128), lambda i: (0, 0))
    return pl.pallas_call(
        body,
        grid=(grid,),
        in_specs=[eblk, eblk, eblk, wblk, wblk, wblk, rblk,
                  pl.BlockSpec((8, 128), lambda i: (0, 0)), rblk],
        out_specs=[eblk, eblk, eblk],
        out_shape=[
            jax.ShapeDtypeStruct((EP, 128), F32),
            jax.ShapeDtypeStruct((EP, 128), F32),
            jax.ShapeDtypeStruct((EP, 128), F32),
        ],
    )(ab, rs, rd, bdj, bdwre, bdwo0r, bemb, ucoef, lmask)


def _tc_node_mid(nap, t0p, bdw2, b2t, bdwblo, bdwbhi, bdwd, bdt, wtpk):
    def body(nap_ref, t0p_ref, bdw2_ref, b2_ref, wblo_ref, wbhi_ref,
             wd0_ref, bd0_ref, wd1_ref, bd1_ref, wd2_ref, bd2_ref, wt_ref,
             dlo_ref, dhi_ref, p0_ref):
        na = nap_ref[0] + nap_ref[1]
        c2 = _swish(jnp.dot(na, bdw2_ref[...], preferred_element_type=F32)
                    + b2_ref[...])
        dlo_ref[...] = jnp.dot(c2, wblo_ref[...], preferred_element_type=F32)
        dhi_ref[...] = jnp.dot(c2, wbhi_ref[...], preferred_element_type=F32)
        t = t0p_ref[0] + t0p_ref[1]
        t = _swish(jnp.dot(t, wd0_ref[...], preferred_element_type=F32) + bd0_ref[...])
        t = _swish(jnp.dot(t, wd1_ref[...], preferred_element_type=F32) + bd1_ref[...])
        t = _swish(jnp.dot(t, wd2_ref[...], preferred_element_type=F32) + bd2_ref[...])
        p0_ref[...] = jnp.dot(t, wt_ref[...], preferred_element_type=F32)

    grid = NPK // BNP
    nblk = pl.BlockSpec((BNP, 128), lambda i: (i, 0))
    pblk = pl.BlockSpec((NC, BNP, 128), lambda i: (0, i, 0))
    wblk = pl.BlockSpec((128, 128), lambda i: (0, 0))
    rblk = pl.BlockSpec((1, 128), lambda i: (0, 0))
    return pl.pallas_call(
        body,
        grid=(grid,),
        in_specs=[pblk, pblk, wblk, rblk, wblk, wblk,
                  wblk, rblk, wblk, rblk, wblk, rblk, wblk],
        out_specs=[nblk, nblk, nblk],
        out_shape=[
            jax.ShapeDtypeStruct((NPK, 128), F32),
            jax.ShapeDtypeStruct((NPK, 128), F32),
            jax.ShapeDtypeStruct((NPK, 128), F32),
        ],
    )(nap, t0p, bdw2, b2t, bdwblo, bdwbhi,
      bdwd[0], bdt[0], bdwd[1], bdt[1], bdwd[2], bdt[2], wtpk)


def _tc_edge2(m, rbf, dslo, dshi, bdw1, b1t, bdreplo, bdrephi, bdwplo,
              bdwphi, bdwskip, bskipt, bdwo1r):
    def body(m_ref, rbf_ref, dlo_ref, dhi_ref, w1_ref, b1_ref, rlo_ref,
             rhi_ref, plo_ref, phi_ref, wsk_ref, bsk_ref, wo1_ref, g1_ref):
        m = m_ref[...]
        rbf = rbf_ref[...]
        m_ji = _swish(jnp.dot(m, w1_ref[...], preferred_element_type=F32)
                      + b1_ref[...])
        xp = jnp.dot(jnp.dot(rbf, rlo_ref[...], preferred_element_type=F32)
                     * dlo_ref[...], plo_ref[...], preferred_element_type=F32)
        xp = xp + jnp.dot(jnp.dot(rbf, rhi_ref[...], preferred_element_type=F32)
                          * dhi_ref[...], phi_ref[...],
                          preferred_element_type=F32)
        m_new = _swish(jnp.dot(m_ji + xp, wsk_ref[...],
                               preferred_element_type=F32) + bsk_ref[...]) + m
        g1_ref[...] = jnp.dot(rbf, wo1_ref[...],
                              preferred_element_type=F32) * m_new

    grid = EP // BEP
    eblk = pl.BlockSpec((BEP, 128), lambda i: (i, 0))
    wblk = pl.BlockSpec((128, 128), lambda i: (0, 0))
    rblk = pl.BlockSpec((1, 128), lambda i: (0, 0))
    return pl.pallas_call(
        body,
        grid=(grid,),
        in_specs=[eblk, eblk, eblk, eblk, wblk, rblk, wblk, wblk, wblk,
                  wblk, wblk, rblk, wblk],
        out_specs=[eblk],
        out_shape=[jax.ShapeDtypeStruct((EP, 128), F32)],
    )(m, rbf, dslo, dshi, bdw1, b1t, bdreplo, bdrephi, bdwplo, bdwphi,
      bdwskip, bskipt, bdwo1r)[0]


def _tc_node_out(t1p, p0, bdwd, bdt, wtpk):
    def body(t1p_ref, p0_ref, wd0_ref, bd0_ref, wd1_ref, bd1_ref,
             wd2_ref, bd2_ref, wt_ref, p_ref):
        t = t1p_ref[0] + t1p_ref[1]
        t = _swish(jnp.dot(t, wd0_ref[...], preferred_element_type=F32) + bd0_ref[...])
        t = _swish(jnp.dot(t, wd1_ref[...], preferred_element_type=F32) + bd1_ref[...])
        t = _swish(jnp.dot(t, wd2_ref[...], preferred_element_type=F32) + bd2_ref[...])
        p_ref[...] = p0_ref[...] + jnp.dot(t, wt_ref[...],
                                           preferred_element_type=F32)

    grid = NPK // BNP
    nblk = pl.BlockSpec((BNP, 128), lambda i: (i, 0))
    pblk = pl.BlockSpec((NC, BNP, 128), lambda i: (0, i, 0))
    wblk = pl.BlockSpec((128, 128), lambda i: (0, 0))
    rblk = pl.BlockSpec((1, 128), lambda i: (0, 0))
    return pl.pallas_call(
        body,
        grid=(grid,),
        in_specs=[pblk, nblk, wblk, rblk, wblk, rblk, wblk, rblk, wblk],
        out_specs=[nblk],
        out_shape=[jax.ShapeDtypeStruct((NPK, 128), F32)],
    )(t1p, p0, bdwd[0], bdt[0], bdwd[1], bdt[1], bdwd[2], bdt[2], wtpk)[0]


def kernel(Z, R, edge_index, data, atom_emb, W_rbf_emb, W_emb, b_emb,
           Wo0_rbf, Wo0_d0, bo0_d0, Wo0_d1, bo0_d1, Wo0_d2, bo0_d2, Wo0_t,
           W1, b1, W2, b2, Wb, Wproj, Wskip, bskip,
           Wo1_rbf, Wo1_d0, bo1_d0, Wo1_d1, bo1_d1, Wo1_d2, bo1_d2, Wo1_t):
    src = edge_index[0].astype(jnp.int32)
    dst = edge_index[1].astype(jnp.int32)
    z32 = Z.astype(jnp.int32)

    ta = atom_emb @ W_emb[0:EMB]
    tb = atom_emb @ W_emb[EMB:2 * EMB]
    wre = W_rbf_emb @ W_emb[2 * EMB:3 * EMB]
    wbrs = jnp.transpose(Wb, (1, 0, 2)).reshape(EMB, NUM_RADIAL * NUM_BILINEAR)

    bdj = jnp.kron(jnp.eye(4, dtype=F32), jnp.ones((32, 32), F32))
    bdwre = _bd(_pad32(wre))
    bdwo0r = _bd(_pad32(Wo0_rbf))
    bdwo1r = _bd(_pad32(Wo1_rbf))
    bdw1 = _bd(W1)
    bdw2 = _bd(W2)
    bdwskip = _bd(Wskip)
    bdwblo = _bd(wbrs[:, 0:32])
    bdwbhi = _bd(_pad32(wbrs[:, 32:48]))

    replo = np.zeros((32, 32), np.float32)
    rephi = np.zeros((32, 32), np.float32)
    for r in range(4):
        for b in range(NUM_BILINEAR):
            replo[r, r * NUM_BILINEAR + b] = 1.0
    for r in range(4, 6):
        for b in range(NUM_BILINEAR):
            rephi[r, (r - 4) * NUM_BILINEAR + b] = 1.0
    bdreplo = _bd(jnp.asarray(replo))
    bdrephi = _bd(jnp.asarray(rephi))
    bdwplo = _bd(jnp.tile(Wproj, (4, 1)))
    bdwphi = _bd(_pad32(jnp.tile(Wproj, (2, 1))))

    bdwd0 = [_bd(Wo0_d0), _bd(Wo0_d1), _bd(Wo0_d2)]
    bdt0 = [_tile4(bo0_d0), _tile4(bo0_d1), _tile4(bo0_d2)]
    bdwd1 = [_bd(Wo1_d0), _bd(Wo1_d1), _bd(Wo1_d2)]
    bdt1 = [_tile4(bo1_d0), _tile4(bo1_d1), _tile4(bo1_d2)]
    wt0pk = _bd(_pad32(Wo0_t))
    wt1pk = _bd(_pad32(Wo1_t))

    zp = jnp.pad(z32, (0, NP - N))
    r32 = jnp.pad(R.astype(F32), ((0, NP - N), (0, EMB - 3)))

    src2d = jnp.pad(src.reshape(N_CHUNKS_EDGE, CHUNK),
                    ((0, NCH_PAD - N_CHUNKS_EDGE), (0, 0)))
    dst2d = jnp.pad(dst.reshape(N_CHUNKS_EDGE, CHUNK),
                    ((0, NCH_PAD - N_CHUNKS_EDGE), (0, 0)))

    tan, tbn = _sc_node_prep(zp, ta, tb)
    ab, rs, rd = _sc_edge_gather(tan, tbn, r32, src2d, dst2d)

    m, g0, rbf = _tc_edge1(ab.reshape(EP, 128), rs.reshape(EP, 128),
                           rd.reshape(EP, 128),
                           bdj, bdwre, bdwo0r,
                           _tile4(b_emb), _ucoef_table(), _lane_mask())

    nap = _sc_scatter_add(m.reshape(E, EMB), dst)
    t0p = _sc_scatter_add(g0.reshape(E, EMB), dst)

    dlo, dhi, p0 = _tc_node_mid(nap.reshape(NC, NPK, 128),
                                t0p.reshape(NC, NPK, 128),
                                bdw2, _tile4(b2), bdwblo, bdwbhi,
                                bdwd0, bdt0, wt0pk)

    dslo, dshi = _sc_gather_pair(dlo.reshape(NPAD, EMB), dhi.reshape(NPAD, EMB),
                                 src2d)

    g1 = _tc_edge2(m, rbf, dslo.reshape(EP, 128), dshi.reshape(EP, 128),
                   bdw1, _tile4(b1), bdreplo, bdrephi, bdwplo, bdwphi,
                   bdwskip, _tile4(bskip), bdwo1r)

    t1p = _sc_scatter_add(g1.reshape(E, EMB), dst)

    ppk = _tc_node_out(t1p.reshape(NC, NPK, 128), p0, bdwd1, bdt1, wt1pk)
    return ppk.reshape(NPAD, EMB)[:N, :NUM_TARGETS]
